# baseline (speedup 1.0000x reference)
import functools

import jax
import jax.numpy as jnp
from jax import lax
from jax.experimental import pallas as pl
from jax.experimental.pallas import tpu as pltpu
from jax.experimental.pallas import tpu_sc as plsc

N = 10000
NPAD = 10240
E = 160000
D = 256
H = 128
ROWBLK = 512
GRID = NPAD // ROWBLK
NT = 16
RPT = NPAD // NT

K = 80
TILE_E = E // NT
NBLK = TILE_E // K

DEGW = 128
KD = 40
TILE_ED = (E // 2) // NT
NBLKD = TILE_ED // KD


def _zero_rows(buf, nrows, ncol16):
    def body(i, _):
        for j in range(ncol16):
            buf[i, pl.ds(16 * j, 16)] = jnp.zeros((16,), jnp.float32)
        return 0
    lax.fori_loop(0, nrows, body, 0)


@functools.cache
def _make_sc_degree():
    mesh = plsc.VectorSubcoreMesh(core_axis_name="c", subcore_axis_name="s")
    return pl.kernel(
        _sc_degree_body,
        out_type=jax.ShapeDtypeStruct((2, NPAD, DEGW), jnp.float32),
        mesh=mesh,
        scratch_types=[
            pltpu.VMEM_SHARED((NPAD, DEGW), jnp.float32),
            pltpu.VMEM((128, DEGW), jnp.float32),
            pltpu.VMEM((KD, DEGW), jnp.float32),
            pltpu.VMEM((NBLKD, KD), jnp.int32),
            pltpu.SemaphoreType.DMA,
        ],
    )


def _sc_degree_body(dst_hbm, dg_hbm, acc, zbuf, ones_v, dst_all, ssem):
    c = lax.axis_index("c")
    t = lax.axis_index("s")

    _zero_rows(zbuf, 128, DEGW // 16)

    def ones_body(i, _):
        for j in range(DEGW // 16):
            ones_v[i, pl.ds(16 * j, 16)] = jnp.ones((16,), jnp.float32)
        return 0
    lax.fori_loop(0, KD, ones_body, 0)

    pltpu.sync_copy(dst_hbm.at[c * NT + t], dst_all)

    for r in range(RPT // 128):
        pltpu.sync_copy(zbuf, acc.at[pl.ds(t * RPT + r * 128, 128)])
    plsc.subcore_barrier()

    def edge_body(i, _):
        pltpu.async_copy(ones_v, acc.at[dst_all.at[i]], ssem, add=True)
        return 0
    lax.fori_loop(0, NBLKD, edge_body, 0)

    def drain_body(i, _):
        pltpu.make_async_copy(ones_v, acc.at[dst_all.at[i]], ssem).wait()
        return 0
    lax.fori_loop(0, NBLKD, drain_body, 0)

    plsc.subcore_barrier()

    for r in range(RPT // 128):
        off = t * RPT + r * 128
        pltpu.sync_copy(acc.at[pl.ds(off, 128)], zbuf)
        pltpu.sync_copy(zbuf, dg_hbm.at[c, pl.ds(off, 128)])


@functools.cache
def _make_sc_agg():
    mesh = plsc.VectorSubcoreMesh(core_axis_name="c", subcore_axis_name="s")
    return pl.kernel(
        _sc_agg_body,
        out_type=(
            jax.ShapeDtypeStruct((NPAD, H), jnp.float32),
            jax.ShapeDtypeStruct((NPAD, H), jnp.float32),
        ),
        mesh=mesh,
        scratch_types=(
            [pltpu.VMEM_SHARED((NPAD, H), jnp.float32)]
            + [pltpu.VMEM((K, H), jnp.float32)] * 4
            + [pltpu.VMEM((K,), jnp.int32)] * 4
            + [pltpu.VMEM((K,), jnp.int32)] * 4
            + [pltpu.SemaphoreType.DMA] * 8
        ),
    )


def _sc_agg_body(ya_hbm, yb_hbm, src_hbm, dst_hbm, sa_hbm, sb_hbm,
                 acc, *scr):
    rows = scr[0:4]
    srcs = scr[4:8]
    dsts = scr[8:12]
    gsems = scr[12:16]
    ssems = scr[16:20]
    c = lax.axis_index("c")
    t = lax.axis_index("s")

    _zero_rows(rows[0], K, H // 16)
    for r in range(RPT // K):
        pltpu.sync_copy(rows[0], acc.at[pl.ds(t * RPT + r * K, K)])
    plsc.subcore_barrier()

    base = t * TILE_E

    def half(ytab):
        for j in range(1):
            off = base + j * K
            pltpu.sync_copy(src_hbm.at[pl.ds(off, K)], srcs[j])
            pltpu.sync_copy(dst_hbm.at[pl.ds(off, K)], dsts[j])
            pltpu.async_copy(ytab.at[srcs[j]], rows[j], gsems[j])

        def turn(j, r):
            q = (r + 1) % 4
            pltpu.make_async_copy(ytab.at[srcs[r]], rows[r], gsems[r]).wait()
            pltpu.async_copy(rows[r], acc.at[dsts[r]], ssems[r], add=True)

            @pl.when(j + 1 < NBLK)
            def _():
                @pl.when(j >= 3)
                def _():
                    pltpu.make_async_copy(
                        rows[q], acc.at[dsts[q]], ssems[q]).wait()
                off = base + (j + 1) * K
                pltpu.sync_copy(src_hbm.at[pl.ds(off, K)], srcs[q])
                pltpu.sync_copy(dst_hbm.at[pl.ds(off, K)], dsts[q])
                pltpu.async_copy(ytab.at[srcs[q]], rows[q], gsems[q])

        def quad(m, _):
            for r in range(4):
                turn(4 * m + r, r)
            return 0
        lax.fori_loop(0, NBLK // 4, quad, 0)
        turn(NBLK - 1, 0)

        for r in (1, 2, 3, 0):
            pltpu.make_async_copy(rows[r], acc.at[dsts[r]], ssems[r]).wait()

    @pl.when(c == 0)
    def _():
        half(ya_hbm)

    @pl.when(c == 1)
    def _():
        half(yb_hbm)

    plsc.subcore_barrier()

    def copy_out(out_hbm):
        for r in range(RPT // K):
            off = t * RPT + r * K
            pltpu.sync_copy(acc.at[pl.ds(off, K)], rows[0])
            pltpu.sync_copy(rows[0], out_hbm.at[pl.ds(off, K)])

    @pl.when(c == 0)
    def _():
        copy_out(sa_hbm)

    @pl.when(c == 1)
    def _():
        copy_out(sb_hbm)


def _dinv(dga_ref, dgb_ref):
    deg = dga_ref[0, :, :1] + dgb_ref[0, :, :1] + 1.0
    return lax.rsqrt(deg)


def _k1_body(x_ref, w_ref, dga_ref, dgb_ref, ya_ref, yb_ref):
    dinv = _dinv(dga_ref, dgb_ref)
    xw = jnp.dot(x_ref[...], w_ref[...], preferred_element_type=jnp.float32)
    y = xw * dinv
    ya_ref[...] = y[:, :H]
    yb_ref[...] = y[:, H:]


def _kmid_body(sa_ref, sb_ref, ya_ref, yb_ref, dga_ref, dgb_ref, b_ref, w_ref,
               oa_ref, ob_ref):
    dinv = _dinv(dga_ref, dgb_ref)
    hl = jnp.maximum(dinv * (sa_ref[...] + ya_ref[...]) + b_ref[:, :H], 0.0)
    hr = jnp.maximum(dinv * (sb_ref[...] + yb_ref[...]) + b_ref[:, H:], 0.0)
    xw = (jnp.dot(hl, w_ref[:H, :], preferred_element_type=jnp.float32)
          + jnp.dot(hr, w_ref[H:, :], preferred_element_type=jnp.float32))
    y = xw * dinv
    oa_ref[...] = y[:, :H]
    ob_ref[...] = y[:, H:]


def _kfin_body(sa_ref, sb_ref, ya_ref, yb_ref, dga_ref, dgb_ref, b_ref,
               out_ref):
    dinv = _dinv(dga_ref, dgb_ref)
    zl = dinv * (sa_ref[...] + ya_ref[...]) + b_ref[:, :H]
    zr = dinv * (sb_ref[...] + yb_ref[...]) + b_ref[:, H:]
    m = jnp.maximum(jnp.max(zl, axis=1, keepdims=True),
                    jnp.max(zr, axis=1, keepdims=True))
    ssum = (jnp.sum(jnp.exp(zl - m), axis=1, keepdims=True)
            + jnp.sum(jnp.exp(zr - m), axis=1, keepdims=True))
    lse = m + jnp.log(ssum)
    out_ref[:, :H] = zl - lse
    out_ref[:, H:] = zr - lse


def _row_spec(w):
    return pl.BlockSpec((ROWBLK, w), lambda i: (i, 0))


def _deg_spec(half):
    return pl.BlockSpec((1, ROWBLK, DEGW), lambda i, h=half: (h, i, 0))


def _full_spec(shape):
    return pl.BlockSpec(shape, lambda i: (0, 0))


_HALF_OUT = (
    jax.ShapeDtypeStruct((NPAD, H), jnp.float32),
    jax.ShapeDtypeStruct((NPAD, H), jnp.float32),
)

_k1 = pl.pallas_call(
    _k1_body,
    grid=(GRID,),
    in_specs=[_row_spec(D), _full_spec((D, D)), _deg_spec(0), _deg_spec(1)],
    out_specs=[_row_spec(H), _row_spec(H)],
    out_shape=_HALF_OUT,
)

_kmid = pl.pallas_call(
    _kmid_body,
    grid=(GRID,),
    in_specs=[_row_spec(H), _row_spec(H), _row_spec(H), _row_spec(H),
              _deg_spec(0), _deg_spec(1),
              _full_spec((1, D)), _full_spec((D, D))],
    out_specs=[_row_spec(H), _row_spec(H)],
    out_shape=_HALF_OUT,
)

_kfin = pl.pallas_call(
    _kfin_body,
    grid=(GRID,),
    in_specs=[_row_spec(H), _row_spec(H), _row_spec(H), _row_spec(H),
              _deg_spec(0), _deg_spec(1), _full_spec((1, D))],
    out_specs=_row_spec(D),
    out_shape=jax.ShapeDtypeStruct((NPAD, D), jnp.float32),
)


def kernel(x, edge_index, W1, b1, W2, b2, W3, b3):
    src = edge_index[0].astype(jnp.int32)
    dst = edge_index[1].astype(jnp.int32)
    xp = jnp.pad(x, ((0, NPAD - N), (0, 0)))
    b1r = b1.reshape(1, D)
    b2r = b2.reshape(1, D)
    b3r = b3.reshape(1, D)

    dst4 = dst.reshape(2 * NT, NBLKD, KD)

    sc_degree = _make_sc_degree()
    sc_agg = _make_sc_agg()
    dg = sc_degree(dst4)
    ya, yb = _k1(xp, W1, dg, dg)
    sa, sb = sc_agg(ya, yb, src, dst)
    ya, yb = _kmid(sa, sb, ya, yb, dg, dg, b1r, W2)
    sa, sb = sc_agg(ya, yb, src, dst)
    ya, yb = _kmid(sa, sb, ya, yb, dg, dg, b2r, W3)
    sa, sb = sc_agg(ya, yb, src, dst)
    out = _kfin(sa, sb, ya, yb, dg, dg, b3r)
    return out[:N]


# speedup vs baseline: 1.5507x; 1.5507x over previous
import functools

import jax
import jax.numpy as jnp
from jax import lax
from jax.experimental import pallas as pl
from jax.experimental.pallas import tpu as pltpu
from jax.experimental.pallas import tpu_sc as plsc

N = 10000
NPAD = 10240
E = 160000
D = 256
H = 128
ROWBLK = 512
GRID = NPAD // ROWBLK
NT = 16
RPT = NPAD // NT

K = 80
TILE_E = E // NT
NBLK = TILE_E // K

DEGW = 128
KD = 40
TILE_ED = (E // 2) // NT
NBLKD = TILE_ED // KD


def _zero_rows(buf, nrows, ncol16):
    def body(i, _):
        for j in range(ncol16):
            buf[i, pl.ds(16 * j, 16)] = jnp.zeros((16,), jnp.float32)
        return 0
    lax.fori_loop(0, nrows, body, 0)


@functools.cache
def _make_sc_degree():
    mesh = plsc.VectorSubcoreMesh(core_axis_name="c", subcore_axis_name="s")
    return pl.kernel(
        _sc_degree_body,
        out_type=jax.ShapeDtypeStruct((2, NPAD, DEGW), jnp.float32),
        mesh=mesh,
        scratch_types=[
            pltpu.VMEM_SHARED((NPAD, DEGW), jnp.float32),
            pltpu.VMEM((128, DEGW), jnp.float32),
            pltpu.VMEM((KD, DEGW), jnp.float32),
            pltpu.VMEM((NBLKD, KD), jnp.int32),
            pltpu.SemaphoreType.DMA,
        ],
    )


def _sc_degree_body(dst_hbm, dg_hbm, acc, zbuf, ones_v, dst_all, ssem):
    c = lax.axis_index("c")
    t = lax.axis_index("s")

    _zero_rows(zbuf, 128, DEGW // 16)

    def ones_body(i, _):
        for j in range(DEGW // 16):
            ones_v[i, pl.ds(16 * j, 16)] = jnp.ones((16,), jnp.float32)
        return 0
    lax.fori_loop(0, KD, ones_body, 0)

    pltpu.sync_copy(dst_hbm.at[c * NT + t], dst_all)

    for r in range(RPT // 128):
        pltpu.sync_copy(zbuf, acc.at[pl.ds(t * RPT + r * 128, 128)])
    plsc.subcore_barrier()

    def edge_body(i, _):
        pltpu.async_copy(ones_v, acc.at[dst_all.at[i]], ssem, add=True)
        return 0
    lax.fori_loop(0, NBLKD, edge_body, 0)

    def drain_body(i, _):
        pltpu.make_async_copy(ones_v, acc.at[dst_all.at[i]], ssem).wait()
        return 0
    lax.fori_loop(0, NBLKD, drain_body, 0)

    plsc.subcore_barrier()

    for r in range(RPT // 128):
        off = t * RPT + r * 128
        pltpu.sync_copy(acc.at[pl.ds(off, 128)], zbuf)
        pltpu.sync_copy(zbuf, dg_hbm.at[c, pl.ds(off, 128)])


@functools.cache
def _make_sc_agg():
    mesh = plsc.VectorSubcoreMesh(core_axis_name="c", subcore_axis_name="s")
    return pl.kernel(
        _sc_agg_body,
        out_type=(
            jax.ShapeDtypeStruct((NPAD, H), jnp.float32),
            jax.ShapeDtypeStruct((NPAD, H), jnp.float32),
        ),
        mesh=mesh,
        scratch_types=(
            [pltpu.VMEM_SHARED((NPAD, H), jnp.float32)]
            + [pltpu.VMEM((K, H), jnp.float32)] * 4
            + [pltpu.VMEM((K,), jnp.int32)] * 4
            + [pltpu.VMEM((K,), jnp.int32)] * 4
            + [pltpu.SemaphoreType.DMA] * 8
        ),
    )


def _sc_agg_body(ya_hbm, yb_hbm, src_hbm, dst_hbm, sa_hbm, sb_hbm,
                 acc, *scr):
    rows = scr[0:4]
    srcs = scr[4:8]
    dsts = scr[8:12]
    gsems = scr[12:16]
    ssems = scr[16:20]
    c = lax.axis_index("c")
    t = lax.axis_index("s")

    _zero_rows(rows[0], K, H // 16)
    for r in range(RPT // K):
        pltpu.sync_copy(rows[0], acc.at[pl.ds(t * RPT + r * K, K)])
    plsc.subcore_barrier()

    base = t * TILE_E

    def half(ytab):
        for j in range(2):
            off = base + j * K
            pltpu.sync_copy(src_hbm.at[pl.ds(off, K)], srcs[j])
            pltpu.sync_copy(dst_hbm.at[pl.ds(off, K)], dsts[j])
            pltpu.async_copy(ytab.at[srcs[j]], rows[j], gsems[j])

        def turn(j, r):
            q = (r + 2) % 4
            pltpu.make_async_copy(ytab.at[srcs[r]], rows[r], gsems[r]).wait()
            pltpu.async_copy(rows[r], acc.at[dsts[r]], ssems[r], add=True)

            @pl.when(j + 2 < NBLK)
            def _():
                @pl.when(j >= 2)
                def _():
                    pltpu.make_async_copy(
                        rows[q], acc.at[dsts[q]], ssems[q]).wait()
                off = base + (j + 2) * K
                pltpu.sync_copy(src_hbm.at[pl.ds(off, K)], srcs[q])
                pltpu.sync_copy(dst_hbm.at[pl.ds(off, K)], dsts[q])
                pltpu.async_copy(ytab.at[srcs[q]], rows[q], gsems[q])

        def quad(m, _):
            for r in range(4):
                turn(4 * m + r, r)
            return 0
        lax.fori_loop(0, NBLK // 4, quad, 0)
        turn(NBLK - 1, 0)

        for r in (1, 2, 3, 0):
            pltpu.make_async_copy(rows[r], acc.at[dsts[r]], ssems[r]).wait()

    @pl.when(c == 0)
    def _():
        half(ya_hbm)

    @pl.when(c == 1)
    def _():
        half(yb_hbm)

    plsc.subcore_barrier()

    def copy_out(out_hbm):
        for r in range(RPT // K):
            off = t * RPT + r * K
            pltpu.sync_copy(acc.at[pl.ds(off, K)], rows[0])
            pltpu.sync_copy(rows[0], out_hbm.at[pl.ds(off, K)])

    @pl.when(c == 0)
    def _():
        copy_out(sa_hbm)

    @pl.when(c == 1)
    def _():
        copy_out(sb_hbm)


def _dinv(dga_ref, dgb_ref):
    deg = dga_ref[0, :, :1] + dgb_ref[0, :, :1] + 1.0
    return lax.rsqrt(deg)


def _k1_body(x_ref, w_ref, dga_ref, dgb_ref, ya_ref, yb_ref):
    dinv = _dinv(dga_ref, dgb_ref)
    xw = jnp.dot(x_ref[...], w_ref[...], preferred_element_type=jnp.float32)
    y = xw * dinv
    ya_ref[...] = y[:, :H]
    yb_ref[...] = y[:, H:]


def _kmid_body(sa_ref, sb_ref, ya_ref, yb_ref, dga_ref, dgb_ref, b_ref, w_ref,
               oa_ref, ob_ref):
    dinv = _dinv(dga_ref, dgb_ref)
    hl = jnp.maximum(dinv * (sa_ref[...] + ya_ref[...]) + b_ref[:, :H], 0.0)
    hr = jnp.maximum(dinv * (sb_ref[...] + yb_ref[...]) + b_ref[:, H:], 0.0)
    xw = (jnp.dot(hl, w_ref[:H, :], preferred_element_type=jnp.float32)
          + jnp.dot(hr, w_ref[H:, :], preferred_element_type=jnp.float32))
    y = xw * dinv
    oa_ref[...] = y[:, :H]
    ob_ref[...] = y[:, H:]


def _kfin_body(sa_ref, sb_ref, ya_ref, yb_ref, dga_ref, dgb_ref, b_ref,
               out_ref):
    dinv = _dinv(dga_ref, dgb_ref)
    zl = dinv * (sa_ref[...] + ya_ref[...]) + b_ref[:, :H]
    zr = dinv * (sb_ref[...] + yb_ref[...]) + b_ref[:, H:]
    m = jnp.maximum(jnp.max(zl, axis=1, keepdims=True),
                    jnp.max(zr, axis=1, keepdims=True))
    ssum = (jnp.sum(jnp.exp(zl - m), axis=1, keepdims=True)
            + jnp.sum(jnp.exp(zr - m), axis=1, keepdims=True))
    lse = m + jnp.log(ssum)
    out_ref[:, :H] = zl - lse
    out_ref[:, H:] = zr - lse


def _row_spec(w):
    return pl.BlockSpec((ROWBLK, w), lambda i: (i, 0))


def _deg_spec(half):
    return pl.BlockSpec((1, ROWBLK, DEGW), lambda i, h=half: (h, i, 0))


def _full_spec(shape):
    return pl.BlockSpec(shape, lambda i: (0, 0))


_HALF_OUT = (
    jax.ShapeDtypeStruct((NPAD, H), jnp.float32),
    jax.ShapeDtypeStruct((NPAD, H), jnp.float32),
)

_k1 = pl.pallas_call(
    _k1_body,
    grid=(GRID,),
    in_specs=[_row_spec(D), _full_spec((D, D)), _deg_spec(0), _deg_spec(1)],
    out_specs=[_row_spec(H), _row_spec(H)],
    out_shape=_HALF_OUT,
)

_kmid = pl.pallas_call(
    _kmid_body,
    grid=(GRID,),
    in_specs=[_row_spec(H), _row_spec(H), _row_spec(H), _row_spec(H),
              _deg_spec(0), _deg_spec(1),
              _full_spec((1, D)), _full_spec((D, D))],
    out_specs=[_row_spec(H), _row_spec(H)],
    out_shape=_HALF_OUT,
)

_kfin = pl.pallas_call(
    _kfin_body,
    grid=(GRID,),
    in_specs=[_row_spec(H), _row_spec(H), _row_spec(H), _row_spec(H),
              _deg_spec(0), _deg_spec(1), _full_spec((1, D))],
    out_specs=_row_spec(D),
    out_shape=jax.ShapeDtypeStruct((NPAD, D), jnp.float32),
)


def kernel(x, edge_index, W1, b1, W2, b2, W3, b3):
    src = edge_index[0].astype(jnp.int32)
    dst = edge_index[1].astype(jnp.int32)
    xp = jnp.pad(x, ((0, NPAD - N), (0, 0)))
    b1r = b1.reshape(1, D)
    b2r = b2.reshape(1, D)
    b3r = b3.reshape(1, D)

    dst4 = dst.reshape(2 * NT, NBLKD, KD)

    sc_degree = _make_sc_degree()
    sc_agg = _make_sc_agg()
    dg = sc_degree(dst4)
    ya, yb = _k1(xp, W1, dg, dg)
    sa, sb = sc_agg(ya, yb, src, dst)
    ya, yb = _kmid(sa, sb, ya, yb, dg, dg, b1r, W2)
    sa, sb = sc_agg(ya, yb, src, dst)
    ya, yb = _kmid(sa, sb, ya, yb, dg, dg, b2r, W3)
    sa, sb = sc_agg(ya, yb, src, dst)
    out = _kfin(sa, sb, ya, yb, dg, dg, b3r)
    return out[:N]


# speedup vs baseline: 1.8596x; 1.1992x over previous
import functools

import jax
import jax.numpy as jnp
from jax import lax
from jax.experimental import pallas as pl
from jax.experimental.pallas import tpu as pltpu
from jax.experimental.pallas import tpu_sc as plsc

N = 10000
NPAD = 10240
E = 160000
D = 256
H = 128
ROWBLK = 512
GRID = NPAD // ROWBLK
NT = 16
RPT = NPAD // NT

K = 80
TILE_E = E // NT
NBLK = TILE_E // K

DEGW = 128
KD = 40
TILE_ED = (E // 2) // NT
NBLKD = TILE_ED // KD


def _zero_rows(buf, nrows, ncol16):
    def body(i, _):
        for j in range(ncol16):
            buf[i, pl.ds(16 * j, 16)] = jnp.zeros((16,), jnp.float32)
        return 0
    lax.fori_loop(0, nrows, body, 0)


@functools.cache
def _make_sc_degree():
    mesh = plsc.VectorSubcoreMesh(core_axis_name="c", subcore_axis_name="s")
    return pl.kernel(
        _sc_degree_body,
        out_type=jax.ShapeDtypeStruct((2, NPAD, DEGW), jnp.float32),
        mesh=mesh,
        scratch_types=[
            pltpu.VMEM_SHARED((NPAD, DEGW), jnp.float32),
            pltpu.VMEM((128, DEGW), jnp.float32),
            pltpu.VMEM((KD, DEGW), jnp.float32),
            pltpu.VMEM((NBLKD, KD), jnp.int32),
            pltpu.SemaphoreType.DMA,
        ],
    )


def _sc_degree_body(dst_hbm, dg_hbm, acc, zbuf, ones_v, dst_all, ssem):
    c = lax.axis_index("c")
    t = lax.axis_index("s")

    _zero_rows(zbuf, 128, DEGW // 16)

    def ones_body(i, _):
        for j in range(DEGW // 16):
            ones_v[i, pl.ds(16 * j, 16)] = jnp.ones((16,), jnp.float32)
        return 0
    lax.fori_loop(0, KD, ones_body, 0)

    pltpu.sync_copy(dst_hbm.at[c * NT + t], dst_all)

    for r in range(RPT // 128):
        pltpu.sync_copy(zbuf, acc.at[pl.ds(t * RPT + r * 128, 128)])
    plsc.subcore_barrier()

    def edge_body(i, _):
        pltpu.async_copy(ones_v, acc.at[dst_all.at[i]], ssem, add=True)
        return 0
    lax.fori_loop(0, NBLKD, edge_body, 0)

    def drain_body(i, _):
        pltpu.make_async_copy(ones_v, acc.at[dst_all.at[i]], ssem).wait()
        return 0
    lax.fori_loop(0, NBLKD, drain_body, 0)

    plsc.subcore_barrier()

    for r in range(RPT // 128):
        off = t * RPT + r * 128
        pltpu.sync_copy(acc.at[pl.ds(off, 128)], zbuf)
        pltpu.sync_copy(zbuf, dg_hbm.at[c, pl.ds(off, 128)])


@functools.cache
def _make_sc_agg():
    mesh = plsc.VectorSubcoreMesh(core_axis_name="c", subcore_axis_name="s")
    return pl.kernel(
        _sc_agg_body,
        out_type=(
            jax.ShapeDtypeStruct((NPAD, H), jnp.float32),
            jax.ShapeDtypeStruct((NPAD, H), jnp.float32),
        ),
        mesh=mesh,
        scratch_types=(
            [pltpu.VMEM_SHARED((NPAD, H), jnp.float32)]
            + [pltpu.VMEM((K, H), jnp.float32)] * 4
            + [pltpu.VMEM((K,), jnp.int32)] * 8
            + [pltpu.VMEM((K,), jnp.int32)] * 8
            + [pltpu.SemaphoreType.DMA] * 16
        ),
    )


def _sc_agg_body(ya_hbm, yb_hbm, src_hbm, dst_hbm, sa_hbm, sb_hbm,
                 acc, *scr):
    rows = scr[0:4]
    srcs = scr[4:12]
    dsts = scr[12:20]
    gsems = scr[20:24]
    ssems = scr[24:28]
    isems = scr[28:36]
    c = lax.axis_index("c")
    t = lax.axis_index("s")

    _zero_rows(rows[0], K, H // 16)
    for r in range(RPT // K):
        pltpu.sync_copy(rows[0], acc.at[pl.ds(t * RPT + r * K, K)])
    plsc.subcore_barrier()

    base = t * TILE_E

    def half(ytab):
        def idx_start(j, w):
            off = base + j * K
            pltpu.async_copy(src_hbm.at[pl.ds(off, K)], srcs[w], isems[w])
            pltpu.async_copy(dst_hbm.at[pl.ds(off, K)], dsts[w], isems[w])

        def idx_wait(j, w):
            off = base + j * K
            pltpu.make_async_copy(
                src_hbm.at[pl.ds(off, K)], srcs[w], isems[w]).wait()
            pltpu.make_async_copy(
                dst_hbm.at[pl.ds(off, K)], dsts[w], isems[w]).wait()

        for jj in range(6):
            idx_start(jj, jj)
        for jj in range(2):
            idx_wait(jj, jj)
            pltpu.async_copy(ytab.at[srcs[jj]], rows[jj], gsems[jj])

        def turn(j, r4, r8):
            q = (r4 + 2) % 4
            p = (r8 + 2) % 8
            w = (r8 + 6) % 8
            pltpu.make_async_copy(ytab.at[srcs[r8]], rows[r4], gsems[r4]).wait()
            pltpu.async_copy(rows[r4], acc.at[dsts[r8]], ssems[r4], add=True)

            @pl.when(j + 2 < NBLK)
            def _():
                @pl.when(j >= 2)
                def _():
                    pltpu.make_async_copy(
                        rows[q], acc.at[dsts[w]], ssems[q]).wait()
                idx_wait(j + 2, p)
                pltpu.async_copy(ytab.at[srcs[p]], rows[q], gsems[q])

                @pl.when(j + 6 < NBLK)
                def _():
                    idx_start(j + 6, w)

        def oct_body(m, _):
            for r in range(8):
                turn(8 * m + r, r % 4, r)
            return 0
        lax.fori_loop(0, NBLK // 8, oct_body, 0)
        for j in range(NBLK - 5, NBLK):
            turn(jnp.int32(j), j % 4, j % 8)

        for r4, r8 in ((1, 1), (2, 2), (3, 3), (0, 4)):
            pltpu.make_async_copy(rows[r4], acc.at[dsts[r8]], ssems[r4]).wait()

    @pl.when(c == 0)
    def _():
        half(ya_hbm)

    @pl.when(c == 1)
    def _():
        half(yb_hbm)

    plsc.subcore_barrier()

    def copy_out(out_hbm):
        for r in range(RPT // K):
            off = t * RPT + r * K
            pltpu.sync_copy(acc.at[pl.ds(off, K)], rows[0])
            pltpu.sync_copy(rows[0], out_hbm.at[pl.ds(off, K)])

    @pl.when(c == 0)
    def _():
        copy_out(sa_hbm)

    @pl.when(c == 1)
    def _():
        copy_out(sb_hbm)


def _dinv(dga_ref, dgb_ref):
    deg = dga_ref[0, :, :1] + dgb_ref[0, :, :1] + 1.0
    return lax.rsqrt(deg)


def _k1_body(x_ref, w_ref, dga_ref, dgb_ref, ya_ref, yb_ref):
    dinv = _dinv(dga_ref, dgb_ref)
    xw = jnp.dot(x_ref[...], w_ref[...], preferred_element_type=jnp.float32)
    y = xw * dinv
    ya_ref[...] = y[:, :H]
    yb_ref[...] = y[:, H:]


def _kmid_body(sa_ref, sb_ref, ya_ref, yb_ref, dga_ref, dgb_ref, b_ref, w_ref,
               oa_ref, ob_ref):
    dinv = _dinv(dga_ref, dgb_ref)
    hl = jnp.maximum(dinv * (sa_ref[...] + ya_ref[...]) + b_ref[:, :H], 0.0)
    hr = jnp.maximum(dinv * (sb_ref[...] + yb_ref[...]) + b_ref[:, H:], 0.0)
    xw = (jnp.dot(hl, w_ref[:H, :], preferred_element_type=jnp.float32)
          + jnp.dot(hr, w_ref[H:, :], preferred_element_type=jnp.float32))
    y = xw * dinv
    oa_ref[...] = y[:, :H]
    ob_ref[...] = y[:, H:]


def _kfin_body(sa_ref, sb_ref, ya_ref, yb_ref, dga_ref, dgb_ref, b_ref,
               out_ref):
    dinv = _dinv(dga_ref, dgb_ref)
    zl = dinv * (sa_ref[...] + ya_ref[...]) + b_ref[:, :H]
    zr = dinv * (sb_ref[...] + yb_ref[...]) + b_ref[:, H:]
    m = jnp.maximum(jnp.max(zl, axis=1, keepdims=True),
                    jnp.max(zr, axis=1, keepdims=True))
    ssum = (jnp.sum(jnp.exp(zl - m), axis=1, keepdims=True)
            + jnp.sum(jnp.exp(zr - m), axis=1, keepdims=True))
    lse = m + jnp.log(ssum)
    out_ref[:, :H] = zl - lse
    out_ref[:, H:] = zr - lse


def _row_spec(w):
    return pl.BlockSpec((ROWBLK, w), lambda i: (i, 0))


def _deg_spec(half):
    return pl.BlockSpec((1, ROWBLK, DEGW), lambda i, h=half: (h, i, 0))


def _full_spec(shape):
    return pl.BlockSpec(shape, lambda i: (0, 0))


_HALF_OUT = (
    jax.ShapeDtypeStruct((NPAD, H), jnp.float32),
    jax.ShapeDtypeStruct((NPAD, H), jnp.float32),
)

_k1 = pl.pallas_call(
    _k1_body,
    grid=(GRID,),
    in_specs=[_row_spec(D), _full_spec((D, D)), _deg_spec(0), _deg_spec(1)],
    out_specs=[_row_spec(H), _row_spec(H)],
    out_shape=_HALF_OUT,
)

_kmid = pl.pallas_call(
    _kmid_body,
    grid=(GRID,),
    in_specs=[_row_spec(H), _row_spec(H), _row_spec(H), _row_spec(H),
              _deg_spec(0), _deg_spec(1),
              _full_spec((1, D)), _full_spec((D, D))],
    out_specs=[_row_spec(H), _row_spec(H)],
    out_shape=_HALF_OUT,
)

_kfin = pl.pallas_call(
    _kfin_body,
    grid=(GRID,),
    in_specs=[_row_spec(H), _row_spec(H), _row_spec(H), _row_spec(H),
              _deg_spec(0), _deg_spec(1), _full_spec((1, D))],
    out_specs=_row_spec(D),
    out_shape=jax.ShapeDtypeStruct((NPAD, D), jnp.float32),
)


def kernel(x, edge_index, W1, b1, W2, b2, W3, b3):
    src = edge_index[0].astype(jnp.int32)
    dst = edge_index[1].astype(jnp.int32)
    xp = jnp.pad(x, ((0, NPAD - N), (0, 0)))
    b1r = b1.reshape(1, D)
    b2r = b2.reshape(1, D)
    b3r = b3.reshape(1, D)

    dst4 = dst.reshape(2 * NT, NBLKD, KD)

    sc_degree = _make_sc_degree()
    sc_agg = _make_sc_agg()
    dg = sc_degree(dst4)
    ya, yb = _k1(xp, W1, dg, dg)
    sa, sb = sc_agg(ya, yb, src, dst)
    ya, yb = _kmid(sa, sb, ya, yb, dg, dg, b1r, W2)
    sa, sb = sc_agg(ya, yb, src, dst)
    ya, yb = _kmid(sa, sb, ya, yb, dg, dg, b2r, W3)
    sa, sb = sc_agg(ya, yb, src, dst)
    out = _kfin(sa, sb, ya, yb, dg, dg, b3r)
    return out[:N]
